# baseline (speedup 1.0000x reference)
import functools

import jax
import jax.numpy as jnp
from jax import lax
from jax.experimental import pallas as pl
from jax.experimental.pallas import tpu as pltpu
from jax.experimental.pallas import tpu_sc as plsc

NC = 2
NS = 16
L = 16
NW = NC * NS

_TOFF = (0, 1, 3, 4, 6, 7, 9, 10, 12, 13)
NF = 10
TR = 8
TLC = 128

CS = 18432
CE = 32768
TCB = 1024


def _sc_loss_body(NA, NB, pred, targ, score, mask, stride, out,
                  pbuf, tbuf, sbuf, mbuf, stbuf, ostage, sem0, sem1):
    nrt = NB // TR
    nhead = CS // TLC
    ntail = (NA - CE + TLC - 1) // TLC
    ncu = nhead + ntail
    NU = nrt * ncu
    SLOTS = ((NU + NW - 1) // NW + 2 + 1) // 2 * 2
    wid = lax.axis_index("s") * NC + lax.axis_index("c")
    u0 = (wid * NU) // NW
    cnt = ((wid + 1) * NU) // NW - u0
    sems = (sem0, sem1)

    def unit_of(k):
        u = u0 + jnp.minimum(k, cnt - 1)
        ucol = u // nrt
        tr = u - ucol * nrt
        tc = jnp.where(ucol < nhead, ucol, ucol + (CE // TLC - nhead))
        return tr * TR, tc * TLC, tc

    def issue(k, b):
        ro, co, _ = unit_of(k)
        sm = sems[b]
        for p in range(NF):
            pltpu.async_copy(pred.at[p, pl.ds(ro, TR), pl.ds(co, TLC)],
                             pbuf.at[b, p], sm)
            pltpu.async_copy(targ.at[_TOFF[p], pl.ds(ro, TR), pl.ds(co, TLC)],
                             tbuf.at[b, p], sm)
        pltpu.async_copy(score.at[pl.ds(ro, TR), pl.ds(co, TLC)], sbuf.at[b], sm)
        pltpu.async_copy(mask.at[pl.ds(ro, TR), pl.ds(co, TLC)], mbuf.at[b], sm)
        pltpu.async_copy(stride.at[pl.ds(co, TLC)], stbuf.at[b], sm)

    def drain(b):
        sm = sems[b]
        pltpu.make_async_copy(
            pred.at[pl.ds(0, NF), pl.ds(0, TR), pl.ds(0, TLC)],
            pbuf.at[b], sm).wait()
        pltpu.make_async_copy(
            targ.at[pl.ds(0, NF), pl.ds(0, TR), pl.ds(0, TLC)],
            tbuf.at[b], sm).wait()
        pltpu.make_async_copy(score.at[pl.ds(0, TR), pl.ds(0, TLC)],
                              sbuf.at[b], sm).wait()
        pltpu.make_async_copy(mask.at[pl.ds(0, TR), pl.ds(0, TLC)],
                              mbuf.at[b], sm).wait()
        pltpu.make_async_copy(stride.at[pl.ds(0, TLC)], stbuf.at[b], sm).wait()

    def compute(k, b, al, an):
        _, _, tc = unit_of(k)
        vc = jnp.where(k < cnt,
                       jnp.minimum(NA - tc * TLC, TLC) // L,
                       0)

        def jbody(ch, carry):
            al, an = carry
            r = ch // vc
            co = (ch - r * vc) * L
            si = 1.0 / stbuf[b, pl.ds(co, L)]
            mk = mbuf[b, r, pl.ds(co, L)]
            w = sbuf[b, r, pl.ds(co, L)] * mk
            an = an + mk
            s = None
            for p in range(0, NF, 2):
                tp = None
                for q in (p, p + 1):
                    pp = pbuf[b, q, r, pl.ds(co, L)]
                    tt = tbuf[b, q, r, pl.ds(co, L)]
                    d = jnp.abs(pp - tt * si)
                    m = jnp.minimum(d, 1.0)
                    t = (d - 0.5 * m) * m
                    tp = t if tp is None else tp + t
                s = tp if s is None else s + tp
            al = al + s * w
            return al, an

        return lax.fori_loop(0, vc * TR, jbody, (al, an))

    issue(jnp.int32(0), 0)

    def pair(kp, carry):
        al, an = carry
        for b in (0, 1):
            k = kp * 2 + b

            @pl.when(k + 1 < SLOTS)
            def _():
                issue(k + 1, 1 - b)

            drain(b)
            al, an = compute(k, b, al, an)
        return al, an

    al = jnp.zeros((L,), jnp.float32)
    an = jnp.zeros((L,), jnp.float32)
    al, an = lax.fori_loop(0, SLOTS // 2, pair, (al, an))

    ostage[pl.ds(0, L)] = al
    ostage[pl.ds(L, L)] = an
    pltpu.sync_copy(ostage, out.at[pl.ds(wid * 2 * L, 2 * L)])


@functools.partial(jax.jit, static_argnums=(5, 6))
def _sc_loss(pred, targ, score, mask, stride, NA, NB):
    mesh = plsc.VectorSubcoreMesh(core_axis_name="c", subcore_axis_name="s",
                                  num_cores=NC, num_subcores=NS)
    body = functools.partial(_sc_loss_body, NA, NB)
    f = pl.kernel(
        body,
        out_type=jax.ShapeDtypeStruct((NW * 2 * L,), jnp.float32),
        mesh=mesh,
        scratch_types=[
            pltpu.VMEM((2, NF, TR, TLC), jnp.float32),
            pltpu.VMEM((2, NF, TR, TLC), jnp.float32),
            pltpu.VMEM((2, TR, TLC), jnp.float32),
            pltpu.VMEM((2, TR, TLC), jnp.float32),
            pltpu.VMEM((2, TLC), jnp.float32),
            pltpu.VMEM((2 * L,), jnp.float32),
            pltpu.SemaphoreType.DMA,
            pltpu.SemaphoreType.DMA,
        ],
        compiler_params=pltpu.CompilerParams(
            needs_layout_passes=False,
            use_tc_tiling_on_sc=True,
            disable_bounds_checks=True,
        ),
    )
    return f(pred, targ, score, mask, stride)


def _tc_body(pred_ref, targ_ref, score_ref, mask_ref, stride_ref,
             lout_ref, nout_ref):
    p = pl.program_id(0)
    i = pl.program_id(1)

    @pl.when((p == 0) & (i == 0))
    def _():
        lout_ref[...] = jnp.zeros_like(lout_ref)
        nout_ref[...] = jnp.zeros_like(nout_ref)

    si = 1.0 / stride_ref[...]
    pr = pred_ref[0]
    tg = targ_ref[0]
    mk = mask_ref[...]
    w = score_ref[...] * mk
    d = jnp.abs(pr - tg * si[None, :])
    m = jnp.minimum(d, 1.0)
    lout_ref[...] += (d - 0.5 * m) * m * w

    @pl.when(p == 0)
    def _():
        nout_ref[...] += mk


@jax.jit
def _tc_loss(pred, targ, score, mask, stride):
    nb = pred.shape[1]
    nblk = (CE - CS) // TCB
    c0 = CS // TCB
    grid = (NF, nblk)
    return pl.pallas_call(
        _tc_body,
        grid=grid,
        in_specs=[
            pl.BlockSpec((1, nb, TCB), lambda p, i: (p, 0, c0 + i)),
            pl.BlockSpec((1, nb, TCB),
                         lambda p, i: ((p // 2) * 3 + p % 2, 0, c0 + i)),
            pl.BlockSpec((nb, TCB), lambda p, i: (0, c0 + i)),
            pl.BlockSpec((nb, TCB), lambda p, i: (0, c0 + i)),
            pl.BlockSpec((TCB,), lambda p, i: (c0 + i,)),
        ],
        out_specs=[
            pl.BlockSpec((nb, TCB), lambda p, i: (0, 0)),
            pl.BlockSpec((nb, TCB), lambda p, i: (0, 0)),
        ],
        out_shape=[
            jax.ShapeDtypeStruct((nb, TCB), jnp.float32),
            jax.ShapeDtypeStruct((nb, TCB), jnp.float32),
        ],
    )(pred, targ, score, mask, stride)


def kernel(pred_kps, target_kps, stride_tensor, target_scores,
           target_scores_sum, fg_mask):
    bs, na = fg_mask.shape

    pred = pred_kps.transpose(2, 0, 1)
    targ = target_kps.transpose(2, 0, 1)
    score = target_scores.reshape(bs, na)
    mask = fg_mask.astype(jnp.float32)
    stride = stride_tensor.reshape(-1)

    o = _sc_loss(pred, targ, score, mask, stride, na, bs).reshape(NW, 2, L)
    tl, tn = _tc_loss(pred, targ, score, mask, stride)
    loss_sum = (o[:, 0].sum() + tl.sum()).reshape(1)
    num_pos = (o[:, 1].sum() + tn.sum()).reshape(1)
    denom = num_pos * 10.0
    safe = jnp.where(denom == 0.0, jnp.float32(1.0), denom)
    l = loss_sum / safe
    ts = target_scores_sum
    lpos = jnp.where(ts == 0.0, l, l / ts)
    return jnp.where(num_pos > 0.0, lpos,
                     jnp.zeros((1,), jnp.float32)).reshape(())


# speedup vs baseline: 2.2070x; 2.2070x over previous
import functools

import jax
import jax.numpy as jnp
from jax import lax
from jax.experimental import pallas as pl
from jax.experimental.pallas import tpu as pltpu
from jax.experimental.pallas import tpu_sc as plsc

NC = 2
NS = 16
L = 16
NW = NC * NS

_TOFF = (0, 1, 3, 4, 6, 7, 9, 10, 12, 13)
NF = 10
TR = 8
TLC = 128

CS = 18432
CE = 32768
TCB = 1024


def _sc_loss_body(NA, NB, pred, targ, score, mask, stride, out,
                  pbuf, tbuf, sbuf, mbuf, stbuf, ostage, sem0, sem1):
    nrt = NB // TR
    nhead = CS // TLC
    ntail = (NA - CE + TLC - 1) // TLC
    ncu = nhead + ntail
    NU = nrt * ncu
    SLOTS = ((NU + NW - 1) // NW + 2 + 1) // 2 * 2
    wid = lax.axis_index("s") * NC + lax.axis_index("c")
    u0 = (wid * NU) // NW
    cnt = ((wid + 1) * NU) // NW - u0
    sems = (sem0, sem1)

    def unit_of(k):
        u = u0 + jnp.minimum(k, cnt - 1)
        ucol = u // nrt
        tr = u - ucol * nrt
        tc = jnp.where(ucol < nhead, ucol, ucol + (CE // TLC - nhead))
        return tr * TR, tc * TLC, tc

    def issue(k, b):
        ro, co, _ = unit_of(k)
        sm = sems[b]
        for p in range(NF):
            pltpu.async_copy(pred.at[p, pl.ds(ro, TR), pl.ds(co, TLC)],
                             pbuf.at[b, p], sm)
            pltpu.async_copy(targ.at[_TOFF[p], pl.ds(ro, TR), pl.ds(co, TLC)],
                             tbuf.at[b, p], sm)
        pltpu.async_copy(score.at[pl.ds(ro, TR), pl.ds(co, TLC)], sbuf.at[b], sm)
        pltpu.async_copy(mask.at[pl.ds(ro, TR), pl.ds(co, TLC)], mbuf.at[b], sm)
        pltpu.async_copy(stride.at[pl.ds(co, TLC)], stbuf.at[b], sm)

    def drain(b):
        sm = sems[b]
        pltpu.make_async_copy(
            pred.at[pl.ds(0, NF), pl.ds(0, TR), pl.ds(0, TLC)],
            pbuf.at[b], sm).wait()
        pltpu.make_async_copy(
            targ.at[pl.ds(0, NF), pl.ds(0, TR), pl.ds(0, TLC)],
            tbuf.at[b], sm).wait()
        pltpu.make_async_copy(score.at[pl.ds(0, TR), pl.ds(0, TLC)],
                              sbuf.at[b], sm).wait()
        pltpu.make_async_copy(mask.at[pl.ds(0, TR), pl.ds(0, TLC)],
                              mbuf.at[b], sm).wait()
        pltpu.make_async_copy(stride.at[pl.ds(0, TLC)], stbuf.at[b], sm).wait()

    def compute(k, b, al, an):
        _, _, tc = unit_of(k)
        vc = jnp.where(k < cnt,
                       jnp.minimum(NA - tc * TLC, TLC) // L,
                       0)

        def jbody(ch, carry):
            al, an = carry
            r = ch // vc
            co = (ch - r * vc) * L
            si = 1.0 / stbuf[b, pl.ds(co, L)]
            mk = mbuf[b, r, pl.ds(co, L)]
            w = sbuf[b, r, pl.ds(co, L)] * mk
            an = an + mk
            s = None
            for p in range(0, NF, 2):
                tp = None
                for q in (p, p + 1):
                    pp = pbuf[b, q, r, pl.ds(co, L)]
                    tt = tbuf[b, q, r, pl.ds(co, L)]
                    d = jnp.abs(pp - tt * si)
                    m = jnp.minimum(d, 1.0)
                    t = (d - 0.5 * m) * m
                    tp = t if tp is None else tp + t
                s = tp if s is None else s + tp
            al = al + s * w
            return al, an

        return lax.fori_loop(0, vc * TR, jbody, (al, an))

    issue(jnp.int32(0), 0)

    def pair(kp, carry):
        al, an = carry
        for b in (0, 1):
            k = kp * 2 + b

            @pl.when(k + 1 < SLOTS)
            def _():
                issue(k + 1, 1 - b)

            drain(b)
            al, an = compute(k, b, al, an)
        return al, an

    al = jnp.zeros((L,), jnp.float32)
    an = jnp.zeros((L,), jnp.float32)
    al, an = lax.fori_loop(0, SLOTS // 2, pair, (al, an))

    ostage[pl.ds(0, L)] = al
    ostage[pl.ds(L, L)] = an
    pltpu.sync_copy(ostage, out.at[pl.ds(wid * 2 * L, 2 * L)])


@functools.partial(jax.jit, static_argnums=(5, 6))
def _sc_loss(pred, targ, score, mask, stride, NA, NB):
    mesh = plsc.VectorSubcoreMesh(core_axis_name="c", subcore_axis_name="s",
                                  num_cores=NC, num_subcores=NS)
    body = functools.partial(_sc_loss_body, NA, NB)
    f = pl.kernel(
        body,
        out_type=jax.ShapeDtypeStruct((NW * 2 * L,), jnp.float32),
        mesh=mesh,
        scratch_types=[
            pltpu.VMEM((2, NF, TR, TLC), jnp.float32),
            pltpu.VMEM((2, NF, TR, TLC), jnp.float32),
            pltpu.VMEM((2, TR, TLC), jnp.float32),
            pltpu.VMEM((2, TR, TLC), jnp.float32),
            pltpu.VMEM((2, TLC), jnp.float32),
            pltpu.VMEM((2 * L,), jnp.float32),
            pltpu.SemaphoreType.DMA,
            pltpu.SemaphoreType.DMA,
        ],
        compiler_params=pltpu.CompilerParams(
            needs_layout_passes=False,
            use_tc_tiling_on_sc=True,
            disable_bounds_checks=True,
        ),
    )
    return f(pred, targ, score, mask, stride)


def _tc_body(pred_ref, targ_ref, score_ref, mask_ref, stride_ref,
             lout_ref, nout_ref):
    i = pl.program_id(0)

    @pl.when(i == 0)
    def _():
        lout_ref[...] = jnp.zeros_like(lout_ref)
        nout_ref[...] = jnp.zeros_like(nout_ref)

    si = 1.0 / stride_ref[...]
    mk = mask_ref[...]
    w = score_ref[...] * mk
    acc = None
    for f in range(NF):
        pr = pred_ref[f]
        tg = targ_ref[_TOFF[f]]
        d = jnp.abs(pr - tg * si[None, :])
        m = jnp.minimum(d, 1.0)
        t = (d - 0.5 * m) * m
        acc = t if acc is None else acc + t
    lout_ref[...] += acc * w
    nout_ref[...] += mk


@jax.jit
def _tc_loss(pred, targ, score, mask, stride):
    nb = pred.shape[1]
    nblk = (CE - CS) // TCB
    c0 = CS // TCB
    return pl.pallas_call(
        _tc_body,
        grid=(nblk,),
        in_specs=[
            pl.BlockSpec((NF, nb, TCB), lambda i: (0, 0, c0 + i)),
            pl.BlockSpec((15, nb, TCB), lambda i: (0, 0, c0 + i)),
            pl.BlockSpec((nb, TCB), lambda i: (0, c0 + i)),
            pl.BlockSpec((nb, TCB), lambda i: (0, c0 + i)),
            pl.BlockSpec((TCB,), lambda i: (c0 + i,)),
        ],
        out_specs=[
            pl.BlockSpec((nb, TCB), lambda i: (0, 0)),
            pl.BlockSpec((nb, TCB), lambda i: (0, 0)),
        ],
        out_shape=[
            jax.ShapeDtypeStruct((nb, TCB), jnp.float32),
            jax.ShapeDtypeStruct((nb, TCB), jnp.float32),
        ],
    )(pred, targ, score, mask, stride)


def kernel(pred_kps, target_kps, stride_tensor, target_scores,
           target_scores_sum, fg_mask):
    bs, na = fg_mask.shape

    pred = pred_kps.transpose(2, 0, 1)
    targ = target_kps.transpose(2, 0, 1)
    score = target_scores.reshape(bs, na)
    mask = fg_mask.astype(jnp.float32)
    stride = stride_tensor.reshape(-1)

    o = _sc_loss(pred, targ, score, mask, stride, na, bs).reshape(NW, 2, L)
    tl, tn = _tc_loss(pred, targ, score, mask, stride)
    loss_sum = (o[:, 0].sum() + tl.sum()).reshape(1)
    num_pos = (o[:, 1].sum() + tn.sum()).reshape(1)
    denom = num_pos * 10.0
    safe = jnp.where(denom == 0.0, jnp.float32(1.0), denom)
    l = loss_sum / safe
    ts = target_scores_sum
    lpos = jnp.where(ts == 0.0, l, l / ts)
    return jnp.where(num_pos > 0.0, lpos,
                     jnp.zeros((1,), jnp.float32)).reshape(())


# speedup vs baseline: 2.2862x; 1.0359x over previous
import functools

import jax
import jax.numpy as jnp
from jax import lax
from jax.experimental import pallas as pl
from jax.experimental.pallas import tpu as pltpu
from jax.experimental.pallas import tpu_sc as plsc

NC = 2
NS = 16
L = 16
NW = NC * NS

_TOFF = (0, 1, 3, 4, 6, 7, 9, 10, 12, 13)
NF = 10
TR = 8
TLC = 128

CS = 16384
CE = 32768
TCB = 1024


def _sc_loss_body(NA, NB, pred, targ, score, mask, stride, out,
                  pbuf, tbuf, sbuf, mbuf, stbuf, ostage, sem0, sem1):
    nrt = NB // TR
    nhead = CS // TLC
    ntail = (NA - CE + TLC - 1) // TLC
    ncu = nhead + ntail
    NU = nrt * ncu
    SLOTS = ((NU + NW - 1) // NW + 1) // 2 * 2
    wid = lax.axis_index("s") * NC + lax.axis_index("c")
    u0 = (wid * NU) // NW
    cnt = ((wid + 1) * NU) // NW - u0
    sems = (sem0, sem1)

    def unit_of(k):
        u = u0 + jnp.minimum(k, cnt - 1)
        ucol = u // nrt
        tr = u - ucol * nrt
        tc = jnp.where(ucol < nhead, ucol, ucol + (CE // TLC - nhead))
        return tr * TR, tc * TLC, tc

    def issue(k, b):
        ro, co, _ = unit_of(k)
        sm = sems[b]
        for p in range(NF):
            pltpu.async_copy(pred.at[p, pl.ds(ro, TR), pl.ds(co, TLC)],
                             pbuf.at[b, p], sm)
            pltpu.async_copy(targ.at[_TOFF[p], pl.ds(ro, TR), pl.ds(co, TLC)],
                             tbuf.at[b, p], sm)
        pltpu.async_copy(score.at[pl.ds(ro, TR), pl.ds(co, TLC)], sbuf.at[b], sm)
        pltpu.async_copy(mask.at[pl.ds(ro, TR), pl.ds(co, TLC)], mbuf.at[b], sm)
        pltpu.async_copy(stride.at[pl.ds(co, TLC)], stbuf.at[b], sm)

    def drain(b):
        sm = sems[b]
        pltpu.make_async_copy(
            pred.at[pl.ds(0, NF), pl.ds(0, TR), pl.ds(0, TLC)],
            pbuf.at[b], sm).wait()
        pltpu.make_async_copy(
            targ.at[pl.ds(0, NF), pl.ds(0, TR), pl.ds(0, TLC)],
            tbuf.at[b], sm).wait()
        pltpu.make_async_copy(score.at[pl.ds(0, TR), pl.ds(0, TLC)],
                              sbuf.at[b], sm).wait()
        pltpu.make_async_copy(mask.at[pl.ds(0, TR), pl.ds(0, TLC)],
                              mbuf.at[b], sm).wait()
        pltpu.make_async_copy(stride.at[pl.ds(0, TLC)], stbuf.at[b], sm).wait()

    def compute(k, b, al, an):
        _, _, tc = unit_of(k)
        vc = jnp.where(k < cnt,
                       jnp.minimum(NA - tc * TLC, TLC) // L,
                       0)

        def jbody(ch, carry):
            al, an = carry
            r = ch // vc
            co = (ch - r * vc) * L
            si = 1.0 / stbuf[b, pl.ds(co, L)]
            mk = mbuf[b, r, pl.ds(co, L)]
            w = sbuf[b, r, pl.ds(co, L)] * mk
            an = an + mk
            s = None
            for p in range(0, NF, 2):
                tp = None
                for q in (p, p + 1):
                    pp = pbuf[b, q, r, pl.ds(co, L)]
                    tt = tbuf[b, q, r, pl.ds(co, L)]
                    d = jnp.abs(pp - tt * si)
                    m = jnp.minimum(d, 1.0)
                    t = (d - 0.5 * m) * m
                    tp = t if tp is None else tp + t
                s = tp if s is None else s + tp
            al = al + s * w
            return al, an

        return lax.fori_loop(0, vc * TR, jbody, (al, an))

    issue(jnp.int32(0), 0)

    def pair(kp, carry):
        al, an = carry
        for b in (0, 1):
            k = kp * 2 + b

            @pl.when(k + 1 < cnt)
            def _():
                issue(k + 1, 1 - b)

            @pl.when(k < cnt)
            def _():
                drain(b)

            al, an = compute(k, b, al, an)
        return al, an

    al = jnp.zeros((L,), jnp.float32)
    an = jnp.zeros((L,), jnp.float32)
    al, an = lax.fori_loop(0, SLOTS // 2, pair, (al, an))

    ostage[pl.ds(0, L)] = al
    ostage[pl.ds(L, L)] = an
    pltpu.sync_copy(ostage, out.at[pl.ds(wid * 2 * L, 2 * L)])


@functools.partial(jax.jit, static_argnums=(5, 6))
def _sc_loss(pred, targ, score, mask, stride, NA, NB):
    mesh = plsc.VectorSubcoreMesh(core_axis_name="c", subcore_axis_name="s",
                                  num_cores=NC, num_subcores=NS)
    body = functools.partial(_sc_loss_body, NA, NB)
    f = pl.kernel(
        body,
        out_type=jax.ShapeDtypeStruct((NW * 2 * L,), jnp.float32),
        mesh=mesh,
        scratch_types=[
            pltpu.VMEM((2, NF, TR, TLC), jnp.float32),
            pltpu.VMEM((2, NF, TR, TLC), jnp.float32),
            pltpu.VMEM((2, TR, TLC), jnp.float32),
            pltpu.VMEM((2, TR, TLC), jnp.float32),
            pltpu.VMEM((2, TLC), jnp.float32),
            pltpu.VMEM((2 * L,), jnp.float32),
            pltpu.SemaphoreType.DMA,
            pltpu.SemaphoreType.DMA,
        ],
        compiler_params=pltpu.CompilerParams(
            needs_layout_passes=False,
            use_tc_tiling_on_sc=True,
            disable_bounds_checks=True,
        ),
    )
    return f(pred, targ, score, mask, stride)


def _tc_body(pred_ref, *rest):
    targ_refs = rest[:NF]
    score_ref, mask_ref, stride_ref, lout_ref, nout_ref = rest[NF:]
    i = pl.program_id(0)

    @pl.when(i == 0)
    def _():
        lout_ref[...] = jnp.zeros_like(lout_ref)
        nout_ref[...] = jnp.zeros_like(nout_ref)

    si = 1.0 / stride_ref[...]
    mk = mask_ref[...]
    w = score_ref[...] * mk
    acc = None
    for f in range(NF):
        pr = pred_ref[f]
        tg = targ_refs[f][0]
        d = jnp.abs(pr - tg * si[None, :])
        m = jnp.minimum(d, 1.0)
        t = (d - 0.5 * m) * m
        acc = t if acc is None else acc + t
    lout_ref[...] += acc * w
    nout_ref[...] += mk


@jax.jit
def _tc_loss(pred, targ, score, mask, stride):
    nb = pred.shape[1]
    nblk = (CE - CS) // TCB
    c0 = CS // TCB
    return pl.pallas_call(
        _tc_body,
        grid=(nblk,),
        in_specs=[
            pl.BlockSpec((NF, nb, TCB), lambda i: (0, 0, c0 + i)),
        ] + [
            pl.BlockSpec((1, nb, TCB),
                         functools.partial(lambda t, i: (t, 0, c0 + i),
                                           _TOFF[f]))
            for f in range(NF)
        ] + [
            pl.BlockSpec((nb, TCB), lambda i: (0, c0 + i)),
            pl.BlockSpec((nb, TCB), lambda i: (0, c0 + i)),
            pl.BlockSpec((TCB,), lambda i: (c0 + i,)),
        ],
        out_specs=[
            pl.BlockSpec((nb, TCB), lambda i: (0, 0)),
            pl.BlockSpec((nb, TCB), lambda i: (0, 0)),
        ],
        out_shape=[
            jax.ShapeDtypeStruct((nb, TCB), jnp.float32),
            jax.ShapeDtypeStruct((nb, TCB), jnp.float32),
        ],
    )(pred, *([targ] * NF), score, mask, stride)


def kernel(pred_kps, target_kps, stride_tensor, target_scores,
           target_scores_sum, fg_mask):
    bs, na = fg_mask.shape

    pred = pred_kps.transpose(2, 0, 1)
    targ = target_kps.transpose(2, 0, 1)
    score = target_scores.reshape(bs, na)
    mask = fg_mask.astype(jnp.float32)
    stride = stride_tensor.reshape(-1)

    o = _sc_loss(pred, targ, score, mask, stride, na, bs).reshape(NW, 2, L)
    tl, tn = _tc_loss(pred, targ, score, mask, stride)
    loss_sum = (o[:, 0].sum() + tl.sum()).reshape(1)
    num_pos = (o[:, 1].sum() + tn.sum()).reshape(1)
    denom = num_pos * 10.0
    safe = jnp.where(denom == 0.0, jnp.float32(1.0), denom)
    l = loss_sum / safe
    ts = target_scores_sum
    lpos = jnp.where(ts == 0.0, l, l / ts)
    return jnp.where(num_pos > 0.0, lpos,
                     jnp.zeros((1,), jnp.float32)).reshape(())


# speedup vs baseline: 2.3455x; 1.0260x over previous
import functools

import jax
import jax.numpy as jnp
from jax import lax
from jax.experimental import pallas as pl
from jax.experimental.pallas import tpu as pltpu
from jax.experimental.pallas import tpu_sc as plsc

NC = 2
NS = 16
L = 16
NW = NC * NS

_TOFF = (0, 1, 3, 4, 6, 7, 9, 10, 12, 13)
NF = 10
TR = 8
TLC = 128

CS = 17408
CE = 32768
TCB = 1024


def _sc_loss_body(NA, NB, pred, targ, score, mask, stride, out,
                  pbuf, tbuf, sbuf, mbuf, stbuf, ostage, sem0, sem1):
    nrt = NB // TR
    nhead = CS // TLC
    ntail = (NA - CE + TLC - 1) // TLC
    ncu = nhead + ntail
    NU = nrt * ncu
    SLOTS = ((NU + NW - 1) // NW + 1) // 2 * 2
    wid = lax.axis_index("s") * NC + lax.axis_index("c")
    u0 = (wid * NU) // NW
    cnt = ((wid + 1) * NU) // NW - u0
    sems = (sem0, sem1)

    def unit_of(k):
        u = u0 + jnp.minimum(k, cnt - 1)
        ucol = u // nrt
        tr = u - ucol * nrt
        tc = jnp.where(ucol < nhead, ucol, ucol + (CE // TLC - nhead))
        return tr * TR, tc * TLC, tc

    def issue(k, b):
        ro, co, _ = unit_of(k)
        sm = sems[b]
        for p in range(NF):
            pltpu.async_copy(pred.at[p, pl.ds(ro, TR), pl.ds(co, TLC)],
                             pbuf.at[b, p], sm)
            pltpu.async_copy(targ.at[_TOFF[p], pl.ds(ro, TR), pl.ds(co, TLC)],
                             tbuf.at[b, p], sm)
        pltpu.async_copy(score.at[pl.ds(ro, TR), pl.ds(co, TLC)], sbuf.at[b], sm)
        pltpu.async_copy(mask.at[pl.ds(ro, TR), pl.ds(co, TLC)], mbuf.at[b], sm)
        pltpu.async_copy(stride.at[pl.ds(co, TLC)], stbuf.at[b], sm)

    def drain(b):
        sm = sems[b]
        pltpu.make_async_copy(
            pred.at[pl.ds(0, NF), pl.ds(0, TR), pl.ds(0, TLC)],
            pbuf.at[b], sm).wait()
        pltpu.make_async_copy(
            targ.at[pl.ds(0, NF), pl.ds(0, TR), pl.ds(0, TLC)],
            tbuf.at[b], sm).wait()
        pltpu.make_async_copy(score.at[pl.ds(0, TR), pl.ds(0, TLC)],
                              sbuf.at[b], sm).wait()
        pltpu.make_async_copy(mask.at[pl.ds(0, TR), pl.ds(0, TLC)],
                              mbuf.at[b], sm).wait()
        pltpu.make_async_copy(stride.at[pl.ds(0, TLC)], stbuf.at[b], sm).wait()

    def compute(k, b, al, an):
        _, _, tc = unit_of(k)
        vc = jnp.where(k < cnt,
                       jnp.minimum(NA - tc * TLC, TLC) // L,
                       0)

        def jbody(ch, carry):
            al, an = carry
            r = ch // vc
            co = (ch - r * vc) * L
            si = 1.0 / stbuf[b, pl.ds(co, L)]
            mk = mbuf[b, r, pl.ds(co, L)]
            w = sbuf[b, r, pl.ds(co, L)] * mk
            an = an + mk
            s = None
            for p in range(0, NF, 2):
                tp = None
                for q in (p, p + 1):
                    pp = pbuf[b, q, r, pl.ds(co, L)]
                    tt = tbuf[b, q, r, pl.ds(co, L)]
                    d = jnp.abs(pp - tt * si)
                    m = jnp.minimum(d, 1.0)
                    t = (d - 0.5 * m) * m
                    tp = t if tp is None else tp + t
                s = tp if s is None else s + tp
            al = al + s * w
            return al, an

        return lax.fori_loop(0, vc * TR, jbody, (al, an))

    issue(jnp.int32(0), 0)

    def pair(kp, carry):
        al, an = carry
        for b in (0, 1):
            k = kp * 2 + b

            @pl.when(k + 1 < cnt)
            def _():
                issue(k + 1, 1 - b)

            @pl.when(k < cnt)
            def _():
                drain(b)

            al, an = compute(k, b, al, an)
        return al, an

    al = jnp.zeros((L,), jnp.float32)
    an = jnp.zeros((L,), jnp.float32)
    al, an = lax.fori_loop(0, SLOTS // 2, pair, (al, an))

    ostage[pl.ds(0, L)] = al
    ostage[pl.ds(L, L)] = an
    pltpu.sync_copy(ostage, out.at[pl.ds(wid * 2 * L, 2 * L)])


@functools.partial(jax.jit, static_argnums=(5, 6))
def _sc_loss(pred, targ, score, mask, stride, NA, NB):
    mesh = plsc.VectorSubcoreMesh(core_axis_name="c", subcore_axis_name="s",
                                  num_cores=NC, num_subcores=NS)
    body = functools.partial(_sc_loss_body, NA, NB)
    f = pl.kernel(
        body,
        out_type=jax.ShapeDtypeStruct((NW * 2 * L,), jnp.float32),
        mesh=mesh,
        scratch_types=[
            pltpu.VMEM((2, NF, TR, TLC), jnp.float32),
            pltpu.VMEM((2, NF, TR, TLC), jnp.float32),
            pltpu.VMEM((2, TR, TLC), jnp.float32),
            pltpu.VMEM((2, TR, TLC), jnp.float32),
            pltpu.VMEM((2, TLC), jnp.float32),
            pltpu.VMEM((2 * L,), jnp.float32),
            pltpu.SemaphoreType.DMA,
            pltpu.SemaphoreType.DMA,
        ],
        compiler_params=pltpu.CompilerParams(
            needs_layout_passes=False,
            use_tc_tiling_on_sc=True,
            disable_bounds_checks=True,
        ),
    )
    return f(pred, targ, score, mask, stride)


def _tc_body(pred_ref, *rest):
    targ_refs = rest[:NF]
    score_ref, mask_ref, stride_ref, lout_ref, nout_ref = rest[NF:]
    i = pl.program_id(0)

    @pl.when(i == 0)
    def _():
        lout_ref[...] = jnp.zeros_like(lout_ref)
        nout_ref[...] = jnp.zeros_like(nout_ref)

    si = 1.0 / stride_ref[...]
    mk = mask_ref[...]
    w = score_ref[...] * mk
    acc = None
    for f in range(NF):
        pr = pred_ref[f]
        tg = targ_refs[f][0]
        d = jnp.abs(pr - tg * si[None, :])
        m = jnp.minimum(d, 1.0)
        t = (d - 0.5 * m) * m
        acc = t if acc is None else acc + t
    lout_ref[...] += acc * w
    nout_ref[...] += mk


@jax.jit
def _tc_loss(pred, targ, score, mask, stride):
    nb = pred.shape[1]
    nblk = (CE - CS) // TCB
    c0 = CS // TCB
    return pl.pallas_call(
        _tc_body,
        grid=(nblk,),
        in_specs=[
            pl.BlockSpec((NF, nb, TCB), lambda i: (0, 0, c0 + i)),
        ] + [
            pl.BlockSpec((1, nb, TCB),
                         functools.partial(lambda t, i: (t, 0, c0 + i),
                                           _TOFF[f]))
            for f in range(NF)
        ] + [
            pl.BlockSpec((nb, TCB), lambda i: (0, c0 + i)),
            pl.BlockSpec((nb, TCB), lambda i: (0, c0 + i)),
            pl.BlockSpec((TCB,), lambda i: (c0 + i,)),
        ],
        out_specs=[
            pl.BlockSpec((nb, TCB), lambda i: (0, 0)),
            pl.BlockSpec((nb, TCB), lambda i: (0, 0)),
        ],
        out_shape=[
            jax.ShapeDtypeStruct((nb, TCB), jnp.float32),
            jax.ShapeDtypeStruct((nb, TCB), jnp.float32),
        ],
    )(pred, *([targ] * NF), score, mask, stride)


def kernel(pred_kps, target_kps, stride_tensor, target_scores,
           target_scores_sum, fg_mask):
    bs, na = fg_mask.shape

    pred = pred_kps.transpose(2, 0, 1)
    targ = target_kps.transpose(2, 0, 1)
    score = target_scores.reshape(bs, na)
    mask = fg_mask.astype(jnp.float32)
    stride = stride_tensor.reshape(-1)

    o = _sc_loss(pred, targ, score, mask, stride, na, bs).reshape(NW, 2, L)
    tl, tn = _tc_loss(pred, targ, score, mask, stride)
    loss_sum = (o[:, 0].sum() + tl.sum()).reshape(1)
    num_pos = (o[:, 1].sum() + tn.sum()).reshape(1)
    denom = num_pos * 10.0
    safe = jnp.where(denom == 0.0, jnp.float32(1.0), denom)
    l = loss_sum / safe
    ts = target_scores_sum
    lpos = jnp.where(ts == 0.0, l, l / ts)
    return jnp.where(num_pos > 0.0, lpos,
                     jnp.zeros((1,), jnp.float32)).reshape(())


# speedup vs baseline: 2.3892x; 1.0186x over previous
import functools

import jax
import jax.numpy as jnp
from jax import lax
from jax.experimental import pallas as pl
from jax.experimental.pallas import tpu as pltpu
from jax.experimental.pallas import tpu_sc as plsc

NC = 2
NS = 16
L = 16
NW = NC * NS

_TOFF = (0, 1, 3, 4, 6, 7, 9, 10, 12, 13)
NF = 10
TR = 8
TLC = 128

CS = 17408
CE = 32768
TCB = 1024


def _sc_loss_body(NA, NB, pred, targ, score, mask, stride, out,
                  pbuf, tbuf, sbuf, mbuf, stbuf, ostage, sem0, sem1):
    nrt = NB // TR
    nhead = CS // TLC
    ntail = (NA - CE + TLC - 1) // TLC
    ncu = nhead + ntail
    NU = nrt * ncu
    SLOTS = ((NU + NW - 1) // NW + 1) // 2 * 2
    wid = lax.axis_index("s") * NC + lax.axis_index("c")
    u0 = (wid * NU) // NW
    cnt = ((wid + 1) * NU) // NW - u0
    sems = (sem0, sem1)

    def unit_of(k):
        u = u0 + jnp.minimum(k, cnt - 1)
        ucol = u // nrt
        tr = u - ucol * nrt
        tc = jnp.where(ucol < nhead, ucol, ucol + (CE // TLC - nhead))
        return tr * TR, tc * TLC, tc

    def issue(k, b):
        ro, co, _ = unit_of(k)
        sm = sems[b]
        for p in range(NF):
            pltpu.async_copy(pred.at[p, pl.ds(ro, TR), pl.ds(co, TLC)],
                             pbuf.at[b, p], sm)
            pltpu.async_copy(targ.at[_TOFF[p], pl.ds(ro, TR), pl.ds(co, TLC)],
                             tbuf.at[b, p], sm)
        pltpu.async_copy(score.at[pl.ds(ro, TR), pl.ds(co, TLC)], sbuf.at[b], sm)
        pltpu.async_copy(mask.at[pl.ds(ro, TR), pl.ds(co, TLC)], mbuf.at[b], sm)
        pltpu.async_copy(stride.at[pl.ds(co, TLC)], stbuf.at[b], sm)

    def drain(b):
        sm = sems[b]
        pltpu.make_async_copy(
            pred.at[pl.ds(0, NF), pl.ds(0, TR), pl.ds(0, TLC)],
            pbuf.at[b], sm).wait()
        pltpu.make_async_copy(
            targ.at[pl.ds(0, NF), pl.ds(0, TR), pl.ds(0, TLC)],
            tbuf.at[b], sm).wait()
        pltpu.make_async_copy(score.at[pl.ds(0, TR), pl.ds(0, TLC)],
                              sbuf.at[b], sm).wait()
        pltpu.make_async_copy(mask.at[pl.ds(0, TR), pl.ds(0, TLC)],
                              mbuf.at[b], sm).wait()
        pltpu.make_async_copy(stride.at[pl.ds(0, TLC)], stbuf.at[b], sm).wait()

    def compute(k, b, al, an):
        _, _, tc = unit_of(k)
        vc = jnp.where(k < cnt,
                       jnp.minimum(NA - tc * TLC, TLC) // L,
                       0)

        def jbody(ch, carry):
            al, an = carry
            r = ch // vc
            co = (ch - r * vc) * L
            si = 1.0 / stbuf[b, pl.ds(co, L)]
            mk = mbuf[b, r, pl.ds(co, L)]
            w = sbuf[b, r, pl.ds(co, L)] * mk
            an = an + mk
            s = None
            for p in range(0, NF, 2):
                tp = None
                for q in (p, p + 1):
                    pp = pbuf[b, q, r, pl.ds(co, L)]
                    tt = tbuf[b, q, r, pl.ds(co, L)]
                    d = jnp.abs(pp - tt * si)
                    m = jnp.minimum(d, 1.0)
                    t = (d - 0.5 * m) * m
                    tp = t if tp is None else tp + t
                s = tp if s is None else s + tp
            al = al + s * w
            return al, an

        return lax.fori_loop(0, vc * TR, jbody, (al, an))

    issue(jnp.int32(0), 0)

    def pair(kp, carry):
        al, an = carry
        for b in (0, 1):
            k = kp * 2 + b

            @pl.when(k + 1 < cnt)
            def _():
                issue(k + 1, 1 - b)

            @pl.when(k < cnt)
            def _():
                drain(b)

            al, an = compute(k, b, al, an)
        return al, an

    al = jnp.zeros((L,), jnp.float32)
    an = jnp.zeros((L,), jnp.float32)
    al, an = lax.fori_loop(0, SLOTS // 2, pair, (al, an))

    ostage[pl.ds(0, L)] = al
    ostage[pl.ds(L, L)] = an
    pltpu.sync_copy(ostage, out.at[pl.ds(wid * 2 * L, 2 * L)])


@functools.partial(jax.jit, static_argnums=(5, 6))
def _sc_loss(pred, targ, score, mask, stride, NA, NB):
    mesh = plsc.VectorSubcoreMesh(core_axis_name="c", subcore_axis_name="s",
                                  num_cores=NC, num_subcores=NS)
    body = functools.partial(_sc_loss_body, NA, NB)
    f = pl.kernel(
        body,
        out_type=jax.ShapeDtypeStruct((NW * 2 * L,), jnp.float32),
        mesh=mesh,
        scratch_types=[
            pltpu.VMEM((2, NF, TR, TLC), jnp.float32),
            pltpu.VMEM((2, NF, TR, TLC), jnp.float32),
            pltpu.VMEM((2, TR, TLC), jnp.float32),
            pltpu.VMEM((2, TR, TLC), jnp.float32),
            pltpu.VMEM((2, TLC), jnp.float32),
            pltpu.VMEM((2 * L,), jnp.float32),
            pltpu.SemaphoreType.DMA,
            pltpu.SemaphoreType.DMA,
        ],
        compiler_params=pltpu.CompilerParams(
            needs_layout_passes=False,
            use_tc_tiling_on_sc=True,
            disable_bounds_checks=True,
        ),
    )
    return f(pred, targ, score, mask, stride)


def _tc_body(nblk, pred_ref, *rest):
    targ_refs = rest[:NF]
    (score_ref, mask_ref, stride_ref, lout_ref, nout_ref,
     lacc_ref, nacc_ref) = rest[NF:]
    i = pl.program_id(0)

    @pl.when(i == 0)
    def _():
        lacc_ref[...] = jnp.zeros_like(lacc_ref)
        nacc_ref[...] = jnp.zeros_like(nacc_ref)

    si = 1.0 / stride_ref[...]
    mk = mask_ref[...]
    w = score_ref[...] * mk
    acc = None
    for f in range(NF):
        pr = pred_ref[f]
        tg = targ_refs[f][0]
        d = jnp.abs(pr - tg * si[None, :])
        m = jnp.minimum(d, 1.0)
        t = (d - 0.5 * m) * m
        acc = t if acc is None else acc + t
    lacc_ref[...] += acc * w
    nacc_ref[...] += mk

    @pl.when(i == nblk - 1)
    def _():
        def fold(x):
            y = x[:8] + x[8:]
            z = None
            for t in range(TCB // 128):
                part = y[:, t * 128:(t + 1) * 128]
                z = part if z is None else z + part
            return z

        lout_ref[...] = fold(lacc_ref[...])
        nout_ref[...] = fold(nacc_ref[...])


@jax.jit
def _tc_loss(pred, targ, score, mask, stride):
    nb = pred.shape[1]
    nblk = (CE - CS) // TCB
    c0 = CS // TCB
    return pl.pallas_call(
        functools.partial(_tc_body, nblk),
        grid=(nblk,),
        in_specs=[
            pl.BlockSpec((NF, nb, TCB), lambda i: (0, 0, c0 + i)),
        ] + [
            pl.BlockSpec((1, nb, TCB),
                         functools.partial(lambda t, i: (t, 0, c0 + i),
                                           _TOFF[f]))
            for f in range(NF)
        ] + [
            pl.BlockSpec((nb, TCB), lambda i: (0, c0 + i)),
            pl.BlockSpec((nb, TCB), lambda i: (0, c0 + i)),
            pl.BlockSpec((TCB,), lambda i: (c0 + i,)),
        ],
        out_specs=[
            pl.BlockSpec((8, 128), lambda i: (0, 0)),
            pl.BlockSpec((8, 128), lambda i: (0, 0)),
        ],
        out_shape=[
            jax.ShapeDtypeStruct((8, 128), jnp.float32),
            jax.ShapeDtypeStruct((8, 128), jnp.float32),
        ],
        scratch_shapes=[
            pltpu.VMEM((nb, TCB), jnp.float32),
            pltpu.VMEM((nb, TCB), jnp.float32),
        ],
    )(pred, *([targ] * NF), score, mask, stride)


def kernel(pred_kps, target_kps, stride_tensor, target_scores,
           target_scores_sum, fg_mask):
    bs, na = fg_mask.shape

    pred = pred_kps.transpose(2, 0, 1)
    targ = target_kps.transpose(2, 0, 1)
    score = target_scores.reshape(bs, na)
    mask = fg_mask.astype(jnp.float32)
    stride = stride_tensor.reshape(-1)

    o = _sc_loss(pred, targ, score, mask, stride, na, bs).reshape(NW, 2, L)
    tl, tn = _tc_loss(pred, targ, score, mask, stride)
    loss_sum = (o[:, 0].sum() + tl.sum()).reshape(1)
    num_pos = (o[:, 1].sum() + tn.sum()).reshape(1)
    denom = num_pos * 10.0
    safe = jnp.where(denom == 0.0, jnp.float32(1.0), denom)
    l = loss_sum / safe
    ts = target_scores_sum
    lpos = jnp.where(ts == 0.0, l, l / ts)
    return jnp.where(num_pos > 0.0, lpos,
                     jnp.zeros((1,), jnp.float32)).reshape(())
